# Initial kernel scaffold; baseline (speedup 1.0000x reference)
import jax
import jax.numpy as jnp
from jax.experimental import pallas as pl


def kernel(movie_title, movie_genres, title_table, genre_table):
    raise NotImplementedError("write your pallas kernel here")


# speedup vs baseline: 2.6716x; 2.6716x over previous
import jax
import jax.numpy as jnp
from jax import lax
from jax.experimental import pallas as pl
from jax.experimental.pallas import tpu as pltpu
from jax.experimental.pallas import tpu_sc as plsc

B = 16384
EMBED = 32
NUM_GENRES = 20
G = 4
NC, NS, L = 2, 16, 16
NW = NC * NS
BPW = B // NW
NBLK = BPW // L


def _sc_body(title_idx, genres, title_table, genre_table, out,
             tidx_v, gidx_v, gtab_v, title_v, genre_v, sem):
    wid = lax.axis_index("s") * NC + lax.axis_index("c")
    base = wid * BPW

    pltpu.sync_copy(title_idx.at[pl.ds(base, BPW)], tidx_v)
    title_cp = pltpu.async_copy(title_table.at[tidx_v], title_v, sem)

    pltpu.sync_copy(genres.at[pl.ds(base * G, BPW * G)], gidx_v)
    pltpu.sync_copy(genre_table, gtab_v)

    quarter = jnp.full((L,), 1.0 / G, jnp.float32)
    spb = L // G

    def block(i, carry):
        ids = gidx_v[pl.ds(i * L, L)]
        for s in range(spb):
            j = i * spb + s
            gid = [ids[s * G + g] for g in range(G)]
            for h in range(EMBED // L):
                sl = pl.ds(h * L, L)
                acc = gtab_v[gid[0], sl]
                for g in range(1, G):
                    acc = acc + gtab_v[gid[g], sl]
                genre_v[j, sl] = acc * quarter
        return carry

    lax.fori_loop(0, BPW // spb, block, 0)

    title_cp.wait()
    pltpu.sync_copy(title_v, out.at[pl.ds(base, BPW), 0])
    pltpu.sync_copy(genre_v, out.at[pl.ds(base, BPW), 1])


_sc_call = pl.kernel(
    _sc_body,
    out_type=jax.ShapeDtypeStruct((B, 2, EMBED), jnp.float32),
    mesh=plsc.VectorSubcoreMesh(core_axis_name="c", subcore_axis_name="s"),
    compiler_params=pltpu.CompilerParams(use_tc_tiling_on_sc=False),
    scratch_types=[
        pltpu.VMEM((BPW,), jnp.int32),
        pltpu.VMEM((BPW * G,), jnp.int32),
        pltpu.VMEM((NUM_GENRES, EMBED), jnp.float32),
        pltpu.VMEM((BPW, EMBED), jnp.float32),
        pltpu.VMEM((BPW, EMBED), jnp.float32),
        pltpu.SemaphoreType.DMA,
    ],
)


@jax.jit
def kernel(movie_title, movie_genres, title_table, genre_table):
    out = _sc_call(movie_title.astype(jnp.int32),
                   movie_genres.astype(jnp.int32).reshape(-1),
                   title_table, genre_table)
    return out.reshape(B, 2 * EMBED)


# speedup vs baseline: 4.4018x; 1.6476x over previous
import jax
import jax.numpy as jnp
from jax import lax
from jax.experimental import pallas as pl
from jax.experimental.pallas import tpu as pltpu
from jax.experimental.pallas import tpu_sc as plsc

B = 16384
EMBED = 32
NUM_GENRES = 20
G = 4
NC, NS, L = 2, 16, 16
NW = NC * NS
CH = 2048
NCHUNK = B // CH


def _sc_body(tidx, gens, tabT, gflat, out,
             row_v, gtab_v, tidx_v, gid_v, otit_v, ogen_v, sem):
    wid = lax.axis_index("s") * NC + lax.axis_index("c")

    row_cp = pltpu.async_copy(tabT.at[wid], row_v, sem)
    pltpu.sync_copy(gflat, gtab_v)
    gbase = wid * NUM_GENRES

    for m in range(NCHUNK):
        c0 = m * CH
        for g in range(G):
            pltpu.sync_copy(gens.at[pl.ds(g * B + c0, CH)], gid_v.at[g])

        def gblk(j, carry):
            sl = pl.ds(j * L, L)
            acc = plsc.load_gather(gtab_v, [gid_v[0, sl] + gbase])
            for g in range(1, G):
                acc = acc + plsc.load_gather(gtab_v, [gid_v[g, sl] + gbase])
            ogen_v[sl] = acc * (1.0 / G)
            return carry

        lax.fori_loop(0, CH // L, gblk, 0)
        pltpu.sync_copy(ogen_v, out.at[pl.ds((EMBED + wid) * B + c0, CH)])

    row_cp.wait()

    for m in range(NCHUNK):
        c0 = m * CH
        pltpu.sync_copy(tidx.at[pl.ds(c0, CH)], tidx_v)

        def tblk(j, carry):
            sl = pl.ds(j * L, L)
            otit_v[sl] = plsc.load_gather(row_v, [tidx_v[sl]])
            return carry

        lax.fori_loop(0, CH // L, tblk, 0)
        pltpu.sync_copy(otit_v, out.at[pl.ds(wid * B + c0, CH)])


_sc_call = pl.kernel(
    _sc_body,
    out_type=jax.ShapeDtypeStruct((2 * EMBED * B,), jnp.float32),
    mesh=plsc.VectorSubcoreMesh(core_axis_name="c", subcore_axis_name="s"),
    compiler_params=pltpu.CompilerParams(use_tc_tiling_on_sc=True,
                                         needs_layout_passes=False),
    scratch_types=[
        pltpu.VMEM((100001,), jnp.float32),
        pltpu.VMEM((NUM_GENRES * EMBED,), jnp.float32),
        pltpu.VMEM((CH,), jnp.int32),
        pltpu.VMEM((G, CH), jnp.int32),
        pltpu.VMEM((CH,), jnp.float32),
        pltpu.VMEM((CH,), jnp.float32),
        pltpu.SemaphoreType.DMA,
    ],
)


@jax.jit
def kernel(movie_title, movie_genres, title_table, genre_table):
    outf = _sc_call(movie_title.astype(jnp.int32),
                    movie_genres.astype(jnp.int32).T.reshape(-1),
                    title_table.T,
                    genre_table.T.reshape(-1))
    return outf.reshape(2 * EMBED, B).T


# speedup vs baseline: 8.8297x; 2.0059x over previous
import jax
import jax.numpy as jnp
from jax import lax
from jax.experimental import pallas as pl
from jax.experimental.pallas import tpu as pltpu
from jax.experimental.pallas import tpu_sc as plsc

B = 16384
EMBED = 32
NUM_GENRES = 20
G = 4
NC, NS, L = 2, 16, 16
NW = NC * NS
SPW = B // NW
CH = 2048
NCHUNK = B // CH


def _sc_body(tidx, gens, tabT, gflat, out,
             row_v, gtab_v, gid_v, ogen_v, tidx_v, otit_v,
             sem_row, sem_stage, sem_out):
    wid = lax.axis_index("s") * NC + lax.axis_index("c")

    row_cp = pltpu.async_copy(tabT.at[wid], row_v, sem_row)

    t_stage = pltpu.async_copy(tidx.at[pl.ds(0, CH)], tidx_v.at[0], sem_stage)

    pltpu.sync_copy(gflat, gtab_v)
    sbase = wid * SPW
    pltpu.sync_copy(gens.at[pl.ds(sbase * G, SPW * G)], gid_v)

    def gblk(j, carry):
        boff = (j // 8) * (G * 128) + (j % 8) * L
        ids = [gid_v[pl.ds(boff + g * 128, L)] for g in range(G)]
        sl = pl.ds(j * L, L)
        for d in range(EMBED):
            acc = plsc.load_gather(gtab_v, [ids[0] + d * NUM_GENRES])
            for g in range(1, G):
                acc = acc + plsc.load_gather(gtab_v, [ids[g] + d * NUM_GENRES])
            ogen_v[d, sl] = acc * (1.0 / G)
        return carry

    lax.fori_loop(0, SPW // L, gblk, 0)
    g_out = pltpu.async_copy(
        ogen_v, out.at[pl.ds(EMBED, EMBED), pl.ds(sbase, SPW)], sem_out)

    row_cp.wait()
    writes = []
    for m in range(NCHUNK):
        t_stage.wait()
        if m + 1 < NCHUNK:
            t_stage = pltpu.async_copy(
                tidx.at[pl.ds((m + 1) * CH, CH)], tidx_v.at[(m + 1) % 2],
                sem_stage)

        def tblk(j, carry):
            sl = pl.ds(j * L, L)
            otit_v[m % 2, sl] = plsc.load_gather(row_v, [tidx_v[m % 2, sl]])
            return carry

        if m >= 2:
            writes[m - 2].wait()
        lax.fori_loop(0, CH // L, tblk, 0)
        writes.append(pltpu.async_copy(
            otit_v.at[m % 2], out.at[wid, pl.ds(m * CH, CH)], sem_out))

    writes[-2].wait()
    writes[-1].wait()
    g_out.wait()


_sc_call = pl.kernel(
    _sc_body,
    out_type=jax.ShapeDtypeStruct((2 * EMBED, B), jnp.float32),
    mesh=plsc.VectorSubcoreMesh(core_axis_name="c", subcore_axis_name="s"),
    compiler_params=pltpu.CompilerParams(use_tc_tiling_on_sc=True,
                                         needs_layout_passes=False),
    scratch_types=[
        pltpu.VMEM((100001,), jnp.float32),
        pltpu.VMEM((NUM_GENRES * EMBED,), jnp.float32),
        pltpu.VMEM((SPW * G,), jnp.int32),
        pltpu.VMEM((EMBED, SPW), jnp.float32),
        pltpu.VMEM((2, CH), jnp.int32),
        pltpu.VMEM((2, CH), jnp.float32),
        pltpu.SemaphoreType.DMA,
        pltpu.SemaphoreType.DMA,
        pltpu.SemaphoreType.DMA,
    ],
)


@jax.jit
def kernel(movie_title, movie_genres, title_table, genre_table):
    gens = (movie_genres.astype(jnp.int32)
            .reshape(B // 128, 128, G).transpose(0, 2, 1).reshape(-1))
    outf = _sc_call(movie_title.astype(jnp.int32), gens,
                    title_table.T, genre_table.T.reshape(-1))
    return outf.T
